# Initial kernel scaffold; baseline (speedup 1.0000x reference)
import jax
import jax.numpy as jnp
from jax.experimental import pallas as pl


def kernel(output, target):
    raise NotImplementedError("write your pallas kernel here")


# speedup vs baseline: 5.5370x; 5.5370x over previous
import jax
import jax.numpy as jnp
from jax.experimental import pallas as pl
from jax.experimental.pallas import tpu as pltpu

_NUM_CLASSES = 80
_NUM_ANCHORS = 5
_AW = (1.3221, 3.19275, 5.05587, 9.47112, 11.2364)
_AH = (1.73145, 4.00944, 8.09892, 4.84053, 10.0071)
_OBJECT_SCALE = 5.0
_SIL_THRESH = 0.6
_MAX_GT = 50
_NB, _NH, _NW = 16, 64, 64
_TPAD = 64

_R_GXL, _R_GXR, _R_GYT, _R_GYB, _R_GAREA, _R_GW, _R_GH, _R_VLD = range(8)
_R_BNV, _R_II, _R_JJ, _R_TX, _R_TY, _R_TW, _R_TH, _R_TCONF, _R_TCLS = range(8, 17)
_N_ROWS = 17


def _anchor_iou(gw, gh, aw, ah):
    mx = jnp.minimum(-0.5 * gw, -0.5 * aw)
    Mx = jnp.maximum(0.5 * gw, 0.5 * aw)
    my = jnp.minimum(-0.5 * gh, -0.5 * ah)
    My = jnp.maximum(0.5 * gh, 0.5 * ah)
    cw = gw + aw - (Mx - mx)
    ch = gh + ah - (My - my)
    inter = jnp.where((cw > 0) & (ch > 0), cw * ch, 0.0)
    union = gw * gh + aw * ah - inter
    return inter / union


def _region_loss_kernel(tgt_s, tgt_v, out_blk, part, sc):
    b = pl.program_id(0)
    a = pl.program_id(1)
    af = a.astype(jnp.float32)
    f32 = jnp.float32

    @pl.when(a == 0)
    def _prepass():
        def body(t, valid_c):
            t0 = tgt_s[b, 0 * _TPAD + t]
            t1 = tgt_s[b, 1 * _TPAD + t]
            t2 = tgt_s[b, 2 * _TPAD + t]
            t3 = tgt_s[b, 3 * _TPAD + t]
            t4 = tgt_s[b, 4 * _TPAD + t]
            valid = valid_c * jnp.where(t1 != 0.0, 1.0, 0.0)
            gx = t1 * _NW
            gy = t2 * _NH
            gw = t3 * _NW
            gh = t4 * _NH
            best = _anchor_iou(gw, gh, _AW[0], _AH[0])
            bn = f32(0.0)
            for k in range(1, _NUM_ANCHORS):
                iou_k = _anchor_iou(gw, gh, _AW[k], _AH[k])
                upd = iou_k > best
                best = jnp.where(upd, iou_k, best)
                bn = jnp.where(upd, f32(k), bn)
            ii = jnp.floor(gx)
            jj = jnp.floor(gy)
            awb = jnp.where(bn == 0, _AW[0], jnp.where(bn == 1, _AW[1],
                  jnp.where(bn == 2, _AW[2], jnp.where(bn == 3, _AW[3], _AW[4]))))
            ahb = jnp.where(bn == 0, _AH[0], jnp.where(bn == 1, _AH[1],
                  jnp.where(bn == 2, _AH[2], jnp.where(bn == 3, _AH[3], _AH[4]))))
            sc[_R_GXL, t] = gx - 0.5 * gw
            sc[_R_GXR, t] = gx + 0.5 * gw
            sc[_R_GYT, t] = gy - 0.5 * gh
            sc[_R_GYB, t] = gy + 0.5 * gh
            sc[_R_GAREA, t] = gw * gh
            sc[_R_GW, t] = gw
            sc[_R_GH, t] = gh
            sc[_R_VLD, t] = valid
            sc[_R_BNV, t] = jnp.where(valid > 0, bn, f32(-1.0))
            sc[_R_II, t] = ii
            sc[_R_JJ, t] = jj
            sc[_R_TX, t] = gx - ii
            sc[_R_TY, t] = gy - jj
            sc[_R_TW, t] = jnp.log(gw / awb)
            sc[_R_TH, t] = jnp.log(gh / ahb)
            sc[_R_TCONF, t] = best
            sc[_R_TCLS, t] = jnp.floor(t0)
            return valid

        jax.lax.fori_loop(0, _MAX_GT, body, f32(1.0))

    aw_a = jnp.where(a == 0, _AW[0], jnp.where(a == 1, _AW[1],
           jnp.where(a == 2, _AW[2], jnp.where(a == 3, _AW[3], _AW[4]))))
    ah_a = jnp.where(a == 0, _AH[0], jnp.where(a == 1, _AH[1],
           jnp.where(a == 2, _AH[2], jnp.where(a == 3, _AH[3], _AH[4]))))

    coli = jax.lax.broadcasted_iota(jnp.int32, (_NH, _NW), 1).astype(f32)
    rowi = jax.lax.broadcasted_iota(jnp.int32, (_NH, _NW), 0).astype(f32)
    sx = jax.nn.sigmoid(out_blk[0, 0])
    sy = jax.nn.sigmoid(out_blk[0, 1])
    pw = jnp.exp(out_blk[0, 2]) * aw_a
    ph = jnp.exp(out_blk[0, 3]) * ah_a
    px = sx + coli
    py = sy + rowi
    pxl = px - 0.5 * pw
    pxr = px + 0.5 * pw
    pyt = py - 0.5 * ph
    pyb = py + 0.5 * ph
    parea = pw * ph

    zeros = jnp.zeros((_NH, _NW), f32)

    def gt_body(t, carry):
        best_iou, maskc, txc, tyc, twc, thc, tcfc, tclc = carry
        gxl = sc[_R_GXL, t]
        gxr = sc[_R_GXR, t]
        gyt = sc[_R_GYT, t]
        gyb = sc[_R_GYB, t]
        garea = sc[_R_GAREA, t]
        gw = sc[_R_GW, t]
        gh = sc[_R_GH, t]
        vld = sc[_R_VLD, t]
        mx = jnp.minimum(pxl, gxl)
        Mx = jnp.maximum(pxr, gxr)
        my = jnp.minimum(pyt, gyt)
        My = jnp.maximum(pyb, gyb)
        cw = pw + gw - (Mx - mx)
        ch = ph + gh - (My - my)
        inter = jnp.where((cw > 0) & (ch > 0), cw * ch, 0.0)
        union = parea + garea - inter
        iou = (inter / union) * vld
        best_iou = jnp.maximum(best_iou, iou)

        ii_m = jnp.where(sc[_R_BNV, t] == af, sc[_R_II, t], f32(-1.0))
        hit = (rowi == sc[_R_JJ, t]) & (coli == ii_m)
        maskc = jnp.where(hit, 1.0, maskc)
        txc = jnp.where(hit, sc[_R_TX, t], txc)
        tyc = jnp.where(hit, sc[_R_TY, t], tyc)
        twc = jnp.where(hit, sc[_R_TW, t], twc)
        thc = jnp.where(hit, sc[_R_TH, t], thc)
        tcfc = jnp.where(hit, sc[_R_TCONF, t], tcfc)
        tclc = jnp.where(hit, sc[_R_TCLS, t], tclc)
        return best_iou, maskc, txc, tyc, twc, thc, tcfc, tclc

    init = (zeros, zeros, zeros, zeros, zeros, zeros, zeros, zeros)
    best_iou, maskc, txc, tyc, twc, thc, tcfc, tclc = jax.lax.fori_loop(
        0, _MAX_GT, gt_body, init)

    conf = jax.nn.sigmoid(out_blk[0, 4])
    rw = out_blk[0, 2]
    rh = out_blk[0, 3]
    lx = jnp.sum(maskc * (sx - txc) ** 2, axis=0, keepdims=True)
    ly = jnp.sum(maskc * (sy - tyc) ** 2, axis=0, keepdims=True)
    lw = jnp.sum(maskc * (rw - twc) ** 2, axis=0, keepdims=True)
    lh = jnp.sum(maskc * (rh - thc) ** 2, axis=0, keepdims=True)
    scale = jnp.where(maskc > 0, _OBJECT_SCALE,
                      jnp.where(best_iou > _SIL_THRESH, 0.0, 1.0))
    dconf = scale * (conf - tcfc)
    lconf = jnp.sum(dconf * dconf, axis=0, keepdims=True)

    lmax = out_blk[0, 5] * maskc
    for c in range(1, _NUM_CLASSES):
        lmax = jnp.maximum(lmax, out_blk[0, 5 + c] * maskc)
    sexp = jnp.zeros((_NH, _NW), f32)
    pick = jnp.zeros((_NH, _NW), f32)
    for c in range(_NUM_CLASSES):
        mc = out_blk[0, 5 + c] * maskc
        sexp = sexp + jnp.exp(mc - lmax)
        pick = jnp.where(tclc == c, mc, pick)
    lse = lmax + jnp.log(sexp)
    lcls = jnp.sum(lse - pick, axis=0, keepdims=True)

    iota_t = jax.lax.broadcasted_iota(jnp.int32, (1, _TPAD), 1).astype(f32)
    t1v = tgt_v[0, 1:2, :]
    fz = jnp.min(jnp.where(t1v == 0.0, iota_t, f32(_TPAD)), axis=1,
                 keepdims=True)
    validv = jnp.where((iota_t < fz) & (a == 0), 1.0, 0.0)

    stacked = jnp.concatenate(
        [lx, ly, lw, lh, lconf, lcls, validv, jnp.zeros((1, _NW), f32)], axis=0)

    @pl.when(a == 0)
    def _init_out():
        part[0] = jnp.zeros((8, _NW), f32)

    part[0] = part[0] + stacked


def kernel(output, target):
    nB = output.shape[0]
    tgt = target.reshape(nB, _MAX_GT, 5).transpose(0, 2, 1)
    tgt = jnp.pad(tgt, ((0, 0), (0, 0), (0, _TPAD - _MAX_GT)))
    tgt_flat = tgt.reshape(nB, 5 * _TPAD)

    part = pl.pallas_call(
        _region_loss_kernel,
        out_shape=jax.ShapeDtypeStruct((nB, 8, _NW), jnp.float32),
        grid=(nB, _NUM_ANCHORS),
        in_specs=[
            pl.BlockSpec(memory_space=pltpu.SMEM),
            pl.BlockSpec((1, 5, _TPAD), lambda b, a: (b, 0, 0)),
            pl.BlockSpec((1, 5 + _NUM_CLASSES, _NH, _NW), lambda b, a: (b, a, 0, 0)),
        ],
        out_specs=pl.BlockSpec((1, 8, _NW), lambda b, a: (b, 0, 0)),
        scratch_shapes=[pltpu.SMEM((_N_ROWS, _TPAD), jnp.float32)],
        compiler_params=pltpu.CompilerParams(
            dimension_semantics=("parallel", "arbitrary"),
        ),
        name="region_loss",
    )(tgt_flat, tgt, output)

    sums = jnp.sum(part, axis=(0, 2))
    ngt = sums[6]
    return (sums[0] + sums[1] + sums[2] + sums[3] + sums[4] + sums[5]) / ngt


# speedup vs baseline: 7.2443x; 1.3083x over previous
import jax
import jax.numpy as jnp
from jax.experimental import pallas as pl
from jax.experimental.pallas import tpu as pltpu

_NUM_CLASSES = 80
_NUM_ANCHORS = 5
_AW = (1.3221, 3.19275, 5.05587, 9.47112, 11.2364)
_AH = (1.73145, 4.00944, 8.09892, 4.84053, 10.0071)
_OBJECT_SCALE = 5.0
_SIL_THRESH = 0.6
_MAX_GT = 50
_NB, _NH, _NW = 16, 64, 64
_TPAD = 64

(_R_GXL, _R_GXR, _R_GYT, _R_GYB, _R_G375, _R_JJ, _R_BNV, _R_II,
 _R_TX, _R_TY, _R_TW, _R_TH, _R_TCONF, _R_TCLS) = range(14)
_N_ROWS = 14


def _anchor_iou(gw, gh, aw, ah):
    mx = jnp.minimum(-0.5 * gw, -0.5 * aw)
    Mx = jnp.maximum(0.5 * gw, 0.5 * aw)
    my = jnp.minimum(-0.5 * gh, -0.5 * ah)
    My = jnp.maximum(0.5 * gh, 0.5 * ah)
    cw = gw + aw - (Mx - mx)
    ch = gh + ah - (My - my)
    inter = jnp.where((cw > 0) & (ch > 0), cw * ch, 0.0)
    union = gw * gh + aw * ah - inter
    return inter / union


def _region_loss_kernel(tgt_s, tgt_v, out_blk, part, sc):
    b = pl.program_id(0)
    a = pl.program_id(1)
    af = a.astype(jnp.float32)
    f32 = jnp.float32

    @pl.when(a == 0)
    def _prepass():
        def body(t, valid_c):
            t0 = tgt_s[b, 0 * _TPAD + t]
            t1 = tgt_s[b, 1 * _TPAD + t]
            t2 = tgt_s[b, 2 * _TPAD + t]
            t3 = tgt_s[b, 3 * _TPAD + t]
            t4 = tgt_s[b, 4 * _TPAD + t]
            valid = valid_c * jnp.where(t1 != 0.0, 1.0, 0.0)
            gx = t1 * _NW
            gy = t2 * _NH
            gw = t3 * _NW
            gh = t4 * _NH
            best = _anchor_iou(gw, gh, _AW[0], _AH[0])
            bn = f32(0.0)
            for k in range(1, _NUM_ANCHORS):
                iou_k = _anchor_iou(gw, gh, _AW[k], _AH[k])
                upd = iou_k > best
                best = jnp.where(upd, iou_k, best)
                bn = jnp.where(upd, f32(k), bn)
            ii = jnp.floor(gx)
            jj = jnp.floor(gy)
            awb = jnp.where(bn == 0, _AW[0], jnp.where(bn == 1, _AW[1],
                  jnp.where(bn == 2, _AW[2], jnp.where(bn == 3, _AW[3], _AW[4]))))
            ahb = jnp.where(bn == 0, _AH[0], jnp.where(bn == 1, _AH[1],
                  jnp.where(bn == 2, _AH[2], jnp.where(bn == 3, _AH[3], _AH[4]))))
            sc[_R_GXL, t] = gx - 0.5 * gw
            sc[_R_GXR, t] = gx + 0.5 * gw
            sc[_R_GYT, t] = gy - 0.5 * gh
            sc[_R_GYB, t] = gy + 0.5 * gh
            sc[_R_G375, t] = jnp.where(valid > 0, 0.375 * gw * gh, f32(jnp.inf))
            sc[_R_JJ, t] = jj
            sc[_R_BNV, t] = jnp.where(valid > 0, bn, f32(-1.0))
            sc[_R_II, t] = ii
            sc[_R_TX, t] = gx - ii
            sc[_R_TY, t] = gy - jj
            sc[_R_TW, t] = jnp.log(gw / awb)
            sc[_R_TH, t] = jnp.log(gh / ahb)
            sc[_R_TCONF, t] = best
            sc[_R_TCLS, t] = jnp.floor(t0)
            return valid

        jax.lax.fori_loop(0, _MAX_GT, body, f32(1.0))

    aw_a = jnp.where(a == 0, _AW[0], jnp.where(a == 1, _AW[1],
           jnp.where(a == 2, _AW[2], jnp.where(a == 3, _AW[3], _AW[4]))))
    ah_a = jnp.where(a == 0, _AH[0], jnp.where(a == 1, _AH[1],
           jnp.where(a == 2, _AH[2], jnp.where(a == 3, _AH[3], _AH[4]))))

    coli = jax.lax.broadcasted_iota(jnp.int32, (_NH, _NW), 1).astype(f32)
    rowi = jax.lax.broadcasted_iota(jnp.int32, (_NH, _NW), 0).astype(f32)
    sx = jax.nn.sigmoid(out_blk[0, 0])
    sy = jax.nn.sigmoid(out_blk[0, 1])
    pw = jnp.exp(out_blk[0, 2]) * aw_a
    ph = jnp.exp(out_blk[0, 3]) * ah_a
    px = sx + coli
    py = sy + rowi
    pxl = px - 0.5 * pw
    pxr = px + 0.5 * pw
    pyt = py - 0.5 * ph
    pyb = py + 0.5 * ph
    parea375 = (0.375 * pw) * ph

    zeros = jnp.zeros((_NH, _NW), f32)
    false_m = zeros > 1.0

    sil = false_m
    maskc = zeros
    txc = zeros
    tyc = zeros
    twc = zeros
    thc = zeros
    tcfc = zeros
    tclc = zeros
    for t in range(_MAX_GT):
        ox = jnp.minimum(pxr, sc[_R_GXR, t]) - jnp.maximum(pxl, sc[_R_GXL, t])
        oy = jnp.minimum(pyb, sc[_R_GYB, t]) - jnp.maximum(pyt, sc[_R_GYT, t])
        inter = jnp.maximum(ox, 0.0) * oy
        sil = sil | (inter > parea375 + sc[_R_G375, t])
        ii_m = jnp.where(sc[_R_BNV, t] == af, sc[_R_II, t], f32(-1.0))
        hit = (rowi == sc[_R_JJ, t]) & (coli == ii_m)
        maskc = jnp.where(hit, 1.0, maskc)
        txc = jnp.where(hit, sc[_R_TX, t], txc)
        tyc = jnp.where(hit, sc[_R_TY, t], tyc)
        twc = jnp.where(hit, sc[_R_TW, t], twc)
        thc = jnp.where(hit, sc[_R_TH, t], thc)
        tcfc = jnp.where(hit, sc[_R_TCONF, t], tcfc)
        tclc = jnp.where(hit, sc[_R_TCLS, t], tclc)

    conf = jax.nn.sigmoid(out_blk[0, 4])
    rw = out_blk[0, 2]
    rh = out_blk[0, 3]
    lx = jnp.sum(maskc * (sx - txc) ** 2, axis=0, keepdims=True)
    ly = jnp.sum(maskc * (sy - tyc) ** 2, axis=0, keepdims=True)
    lw = jnp.sum(maskc * (rw - twc) ** 2, axis=0, keepdims=True)
    lh = jnp.sum(maskc * (rh - thc) ** 2, axis=0, keepdims=True)
    scale = jnp.where(maskc > 0, _OBJECT_SCALE, jnp.where(sil, 0.0, 1.0))
    dconf = scale * (conf - tcfc)
    lconf = jnp.sum(dconf * dconf, axis=0, keepdims=True)

    sexp = jnp.zeros((_NH, _NW), f32)
    pick = jnp.zeros((_NH, _NW), f32)
    for c in range(_NUM_CLASSES):
        mc = out_blk[0, 5 + c] * maskc
        sexp = sexp + jnp.exp(mc)
        pick = jnp.where(tclc == c, mc, pick)
    lse = jnp.log(sexp)
    lcls = jnp.sum(lse - pick, axis=0, keepdims=True)

    iota_t = jax.lax.broadcasted_iota(jnp.int32, (1, _TPAD), 1).astype(f32)
    t1v = tgt_v[0, 1:2, :]
    fz = jnp.min(jnp.where(t1v == 0.0, iota_t, f32(_TPAD)), axis=1,
                 keepdims=True)
    validv = jnp.where((iota_t < fz) & (a == 0), 1.0, 0.0)

    stacked = jnp.concatenate(
        [lx, ly, lw, lh, lconf, lcls, validv, jnp.zeros((1, _NW), f32)], axis=0)

    @pl.when(a == 0)
    def _init_out():
        part[0] = jnp.zeros((8, _NW), f32)

    part[0] = part[0] + stacked


def kernel(output, target):
    nB = output.shape[0]
    tgt = target.reshape(nB, _MAX_GT, 5).transpose(0, 2, 1)
    tgt = jnp.pad(tgt, ((0, 0), (0, 0), (0, _TPAD - _MAX_GT)))
    tgt_flat = tgt.reshape(nB, 5 * _TPAD)

    part = pl.pallas_call(
        _region_loss_kernel,
        out_shape=jax.ShapeDtypeStruct((nB, 8, _NW), jnp.float32),
        grid=(nB, _NUM_ANCHORS),
        in_specs=[
            pl.BlockSpec(memory_space=pltpu.SMEM),
            pl.BlockSpec((1, 5, _TPAD), lambda b, a: (b, 0, 0)),
            pl.BlockSpec((1, 5 + _NUM_CLASSES, _NH, _NW), lambda b, a: (b, a, 0, 0)),
        ],
        out_specs=pl.BlockSpec((1, 8, _NW), lambda b, a: (b, 0, 0)),
        scratch_shapes=[pltpu.SMEM((_N_ROWS, _TPAD), jnp.float32)],
        compiler_params=pltpu.CompilerParams(
            dimension_semantics=("parallel", "arbitrary"),
        ),
        name="region_loss",
    )(tgt_flat, tgt, output)

    sums = jnp.sum(part, axis=(0, 2))
    ngt = sums[6]
    return (sums[0] + sums[1] + sums[2] + sums[3] + sums[4] + sums[5]) / ngt


# speedup vs baseline: 9.6480x; 1.3318x over previous
import jax
import jax.numpy as jnp
from jax.experimental import pallas as pl
from jax.experimental.pallas import tpu as pltpu

_NUM_CLASSES = 80
_NUM_ANCHORS = 5
_AW = (1.3221, 3.19275, 5.05587, 9.47112, 11.2364)
_AH = (1.73145, 4.00944, 8.09892, 4.84053, 10.0071)
_OBJECT_SCALE = 5.0
_MAX_GT = 50
_NB, _NH, _NW = 16, 64, 64
_NHS, _NL = 32, 128
_TPAD = 128

(_R_GXL, _R_GXR, _R_GYT, _R_GYB, _R_G375, _R_JJ, _R_BNV, _R_II,
 _R_TX, _R_TY, _R_TW, _R_TH, _R_TCONF, _R_TCLS) = range(14)
_N_ROWS = 14


def _anchor_iou(gw, gh, aw, ah):
    mx = jnp.minimum(-0.5 * gw, -0.5 * aw)
    Mx = jnp.maximum(0.5 * gw, 0.5 * aw)
    my = jnp.minimum(-0.5 * gh, -0.5 * ah)
    My = jnp.maximum(0.5 * gh, 0.5 * ah)
    cw = gw + aw - (Mx - mx)
    ch = gh + ah - (My - my)
    inter = jnp.where((cw > 0) & (ch > 0), cw * ch, 0.0)
    union = gw * gh + aw * ah - inter
    return inter / union


def _region_loss_kernel(tgt_s, tgt_v, out_blk, part, sc):
    b = pl.program_id(0)
    a = pl.program_id(1)
    af = a.astype(jnp.float32)
    f32 = jnp.float32

    @pl.when(a == 0)
    def _prepass():
        def body(t, valid_c):
            t0 = tgt_s[b, 0 * _TPAD + t]
            t1 = tgt_s[b, 1 * _TPAD + t]
            t2 = tgt_s[b, 2 * _TPAD + t]
            t3 = tgt_s[b, 3 * _TPAD + t]
            t4 = tgt_s[b, 4 * _TPAD + t]
            valid = valid_c * jnp.where(t1 != 0.0, 1.0, 0.0)
            gx = t1 * _NW
            gy = t2 * _NH
            gw = t3 * _NW
            gh = t4 * _NH
            best = _anchor_iou(gw, gh, _AW[0], _AH[0])
            bn = f32(0.0)
            for k in range(1, _NUM_ANCHORS):
                iou_k = _anchor_iou(gw, gh, _AW[k], _AH[k])
                upd = iou_k > best
                best = jnp.where(upd, iou_k, best)
                bn = jnp.where(upd, f32(k), bn)
            ii = jnp.floor(gx)
            jj = jnp.floor(gy)
            awb = jnp.where(bn == 0, _AW[0], jnp.where(bn == 1, _AW[1],
                  jnp.where(bn == 2, _AW[2], jnp.where(bn == 3, _AW[3], _AW[4]))))
            ahb = jnp.where(bn == 0, _AH[0], jnp.where(bn == 1, _AH[1],
                  jnp.where(bn == 2, _AH[2], jnp.where(bn == 3, _AH[3], _AH[4]))))
            sc[_R_GXL, t] = gx - 0.5 * gw
            sc[_R_GXR, t] = gx + 0.5 * gw
            sc[_R_GYT, t] = gy - 0.5 * gh
            sc[_R_GYB, t] = gy + 0.5 * gh
            sc[_R_G375, t] = jnp.where(valid > 0, 0.375 * gw * gh, f32(jnp.inf))
            sc[_R_JJ, t] = jj
            sc[_R_BNV, t] = jnp.where(valid > 0, bn, f32(-1.0))
            sc[_R_II, t] = ii
            sc[_R_TX, t] = gx - ii
            sc[_R_TY, t] = gy - jj
            sc[_R_TW, t] = jnp.log(gw / awb)
            sc[_R_TH, t] = jnp.log(gh / ahb)
            sc[_R_TCONF, t] = best
            sc[_R_TCLS, t] = jnp.floor(t0)
            return valid

        jax.lax.fori_loop(0, _MAX_GT, body, f32(1.0))

    aw_a = jnp.where(a == 0, _AW[0], jnp.where(a == 1, _AW[1],
           jnp.where(a == 2, _AW[2], jnp.where(a == 3, _AW[3], _AW[4]))))
    ah_a = jnp.where(a == 0, _AH[0], jnp.where(a == 1, _AH[1],
           jnp.where(a == 2, _AH[2], jnp.where(a == 3, _AH[3], _AH[4]))))

    li = jax.lax.broadcasted_iota(jnp.int32, (_NHS, _NL), 1)
    si = jax.lax.broadcasted_iota(jnp.int32, (_NHS, _NL), 0)
    coli = (li & 63).astype(f32)
    rowi = (si * 2 + (li >> 6)).astype(f32)
    sx = jax.nn.sigmoid(out_blk[0, 0])
    sy = jax.nn.sigmoid(out_blk[0, 1])
    pw = jnp.exp(out_blk[0, 2]) * aw_a
    ph = jnp.exp(out_blk[0, 3]) * ah_a
    px = sx + coli
    py = sy + rowi
    pxl = px - 0.5 * pw
    pxr = px + 0.5 * pw
    pyt = py - 0.5 * ph
    pyb = py + 0.5 * ph
    parea375 = (0.375 * pw) * ph

    zeros = jnp.zeros((_NHS, _NL), f32)
    false_m = zeros > 1.0

    sil = false_m
    maskc = zeros
    txc = zeros
    tyc = zeros
    twc = zeros
    thc = zeros
    tcfc = zeros
    tclc = zeros
    for t in range(_MAX_GT):
        ox = jnp.minimum(pxr, sc[_R_GXR, t]) - jnp.maximum(pxl, sc[_R_GXL, t])
        oy = jnp.minimum(pyb, sc[_R_GYB, t]) - jnp.maximum(pyt, sc[_R_GYT, t])
        inter = jnp.maximum(ox, 0.0) * oy
        sil = sil | (inter > parea375 + sc[_R_G375, t])
        ii_m = jnp.where(sc[_R_BNV, t] == af, sc[_R_II, t], f32(-1.0))
        hit = (rowi == sc[_R_JJ, t]) & (coli == ii_m)
        maskc = jnp.where(hit, 1.0, maskc)
        txc = jnp.where(hit, sc[_R_TX, t], txc)
        tyc = jnp.where(hit, sc[_R_TY, t], tyc)
        twc = jnp.where(hit, sc[_R_TW, t], twc)
        thc = jnp.where(hit, sc[_R_TH, t], thc)
        tcfc = jnp.where(hit, sc[_R_TCONF, t], tcfc)
        tclc = jnp.where(hit, sc[_R_TCLS, t], tclc)

    conf = jax.nn.sigmoid(out_blk[0, 4])
    rw = out_blk[0, 2]
    rh = out_blk[0, 3]
    lx = jnp.sum(maskc * (sx - txc) ** 2, axis=0, keepdims=True)
    ly = jnp.sum(maskc * (sy - tyc) ** 2, axis=0, keepdims=True)
    lw = jnp.sum(maskc * (rw - twc) ** 2, axis=0, keepdims=True)
    lh = jnp.sum(maskc * (rh - thc) ** 2, axis=0, keepdims=True)
    scale = jnp.where(maskc > 0, _OBJECT_SCALE, jnp.where(sil, 0.0, 1.0))
    dconf = scale * (conf - tcfc)
    lconf = jnp.sum(dconf * dconf, axis=0, keepdims=True)

    sexp = jnp.zeros((_NHS, _NL), f32)
    pick = jnp.zeros((_NHS, _NL), f32)
    for c in range(_NUM_CLASSES):
        mc = out_blk[0, 5 + c] * maskc
        sexp = sexp + jnp.exp(mc)
        pick = jnp.where(tclc == c, mc, pick)
    lse = jnp.log(sexp)
    lcls = jnp.sum(lse - pick, axis=0, keepdims=True)

    iota_t = jax.lax.broadcasted_iota(jnp.int32, (1, _TPAD), 1).astype(f32)
    t1v = tgt_v[0, 1:2, :]
    fz = jnp.min(jnp.where(t1v == 0.0, iota_t, f32(_TPAD)), axis=1,
                 keepdims=True)
    validv = jnp.where((iota_t < fz) & (a == 0), 1.0, 0.0)

    stacked = jnp.concatenate(
        [lx, ly, lw, lh, lconf, lcls, validv, jnp.zeros((1, _NL), f32)], axis=0)

    @pl.when(a == 0)
    def _init_out():
        part[0] = jnp.zeros((8, _NL), f32)

    part[0] = part[0] + stacked


def kernel(output, target):
    nB = output.shape[0]
    out2 = output.reshape(nB, _NUM_ANCHORS * (5 + _NUM_CLASSES), _NHS, _NL)
    tgt = target.reshape(nB, _MAX_GT, 5).transpose(0, 2, 1)
    tgt = jnp.pad(tgt, ((0, 0), (0, 0), (0, _TPAD - _MAX_GT)))
    tgt_flat = tgt.reshape(nB, 5 * _TPAD)

    part = pl.pallas_call(
        _region_loss_kernel,
        out_shape=jax.ShapeDtypeStruct((nB, 8, _NL), jnp.float32),
        grid=(nB, _NUM_ANCHORS),
        in_specs=[
            pl.BlockSpec(memory_space=pltpu.SMEM),
            pl.BlockSpec((1, 5, _TPAD), lambda b, a: (b, 0, 0)),
            pl.BlockSpec((1, 5 + _NUM_CLASSES, _NHS, _NL), lambda b, a: (b, a, 0, 0)),
        ],
        out_specs=pl.BlockSpec((1, 8, _NL), lambda b, a: (b, 0, 0)),
        scratch_shapes=[pltpu.SMEM((_N_ROWS, _TPAD), jnp.float32)],
        compiler_params=pltpu.CompilerParams(
            dimension_semantics=("parallel", "arbitrary"),
        ),
        name="region_loss",
    )(tgt_flat, tgt, out2)

    sums = jnp.sum(part, axis=(0, 2))
    ngt = sums[6]
    return (sums[0] + sums[1] + sums[2] + sums[3] + sums[4] + sums[5]) / ngt


# speedup vs baseline: 10.4406x; 1.0822x over previous
import jax
import jax.numpy as jnp
from jax.experimental import pallas as pl
from jax.experimental.pallas import tpu as pltpu

_NUM_CLASSES = 80
_NUM_ANCHORS = 5
_AW = (1.3221, 3.19275, 5.05587, 9.47112, 11.2364)
_AH = (1.73145, 4.00944, 8.09892, 4.84053, 10.0071)
_OBJECT_SCALE = 5.0
_MAX_GT = 50
_NB, _NH, _NW = 16, 64, 64
_NHS, _NL = 32, 128
_TPAD = 128
_LOG80 = 4.382026634673881

(_R_GXL, _R_GXR, _R_GYT, _R_GYB, _R_G375, _R_JJ, _R_BNV, _R_II,
 _R_TX, _R_TY, _R_TW, _R_TH, _R_TCONF, _R_TCLS, _R_BN) = range(15)
_N_ROWS = 15


def _anchor_iou(gw, gh, aw, ah):
    mx = jnp.minimum(-0.5 * gw, -0.5 * aw)
    Mx = jnp.maximum(0.5 * gw, 0.5 * aw)
    my = jnp.minimum(-0.5 * gh, -0.5 * ah)
    My = jnp.maximum(0.5 * gh, 0.5 * ah)
    cw = gw + aw - (Mx - mx)
    ch = gh + ah - (My - my)
    inter = jnp.where((cw > 0) & (ch > 0), cw * ch, 0.0)
    union = gw * gh + aw * ah - inter
    return inter / union


def _sel_anchor(v, table):
    r = table[-1]
    for k in range(_NUM_ANCHORS - 2, -1, -1):
        r = jnp.where(v == k, table[k], r)
    return r


def _region_loss_kernel(tgt_s, tgt_v, raw_blk, cls_hbm, part,
                        sc, swin, wrow, gbuf, sem_g, sem_w):
    b = pl.program_id(0)
    f32 = jnp.float32

    def body(t, valid_c):
        t0 = tgt_s[b, 0 * _TPAD + t]
        t1 = tgt_s[b, 1 * _TPAD + t]
        t2 = tgt_s[b, 2 * _TPAD + t]
        t3 = tgt_s[b, 3 * _TPAD + t]
        t4 = tgt_s[b, 4 * _TPAD + t]
        valid = valid_c * jnp.where(t1 != 0.0, 1.0, 0.0)
        gx = t1 * _NW
        gy = t2 * _NH
        gw = t3 * _NW
        gh = t4 * _NH
        best = _anchor_iou(gw, gh, _AW[0], _AH[0])
        bn = f32(0.0)
        for k in range(1, _NUM_ANCHORS):
            iou_k = _anchor_iou(gw, gh, _AW[k], _AH[k])
            upd = iou_k > best
            best = jnp.where(upd, iou_k, best)
            bn = jnp.where(upd, f32(k), bn)
        ii = jnp.floor(gx)
        jj = jnp.floor(gy)
        awb = _sel_anchor(bn, _AW)
        ahb = _sel_anchor(bn, _AH)
        sc[_R_GXL, t] = gx - 0.5 * gw
        sc[_R_GXR, t] = gx + 0.5 * gw
        sc[_R_GYT, t] = gy - 0.5 * gh
        sc[_R_GYB, t] = gy + 0.5 * gh
        sc[_R_G375, t] = jnp.where(valid > 0, 0.375 * gw * gh, f32(jnp.inf))
        sc[_R_JJ, t] = jj
        sc[_R_BNV, t] = jnp.where(valid > 0, bn, f32(-1.0))
        sc[_R_II, t] = ii
        sc[_R_TX, t] = gx - ii
        sc[_R_TY, t] = gy - jj
        sc[_R_TW, t] = jnp.log(gw / awb)
        sc[_R_TH, t] = jnp.log(gh / ahb)
        sc[_R_TCONF, t] = best
        sc[_R_TCLS, t] = jnp.floor(t0)
        sc[_R_BN, t] = bn
        return valid

    jax.lax.fori_loop(0, _MAX_GT, body, f32(1.0))

    for t in range(_MAX_GT):
        bn_i = sc[_R_BN, t].astype(jnp.int32)
        jj_i = sc[_R_JJ, t].astype(jnp.int32)
        pltpu.make_async_copy(
            cls_hbm.at[b, bn_i, pl.ds(0, 5 + _NUM_CLASSES), jj_i],
            gbuf.at[t], sem_g).start()

    iota_t = jax.lax.broadcasted_iota(jnp.int32, (1, _TPAD), 1).astype(f32)
    t1v = tgt_v[0, 1:2, :]
    t2v = tgt_v[0, 2:3, :]
    t3v = tgt_v[0, 3:4, :]
    t4v = tgt_v[0, 4:5, :]
    fz = jnp.min(jnp.where(t1v == 0.0, iota_t, f32(_TPAD)), axis=1,
                 keepdims=True)
    validv = jnp.where(iota_t < fz, 1.0, 0.0)
    gwv = t3v * _NW
    ghv = t4v * _NH
    bestv = _anchor_iou(gwv, ghv, _AW[0], _AH[0])
    bnv = jnp.zeros((1, _TPAD), f32)
    for k in range(1, _NUM_ANCHORS):
        iou_k = _anchor_iou(gwv, ghv, _AW[k], _AH[k])
        updv = iou_k > bestv
        bestv = jnp.where(updv, iou_k, bestv)
        bnv = jnp.where(updv, f32(k), bnv)
    iiv = jnp.floor(t1v * _NW)
    jjv = jnp.floor(t2v * _NH)
    keyv = jnp.where(validv > 0, (bnv * 64 + jjv) * 64 + iiv, f32(-1.0))
    ones_r = jnp.ones((1, _TPAD), f32)
    kT = jax.lax.dot_general(keyv, ones_r, (((0,), (0,)), ((), ())),
                             preferred_element_type=f32)
    kB = jnp.broadcast_to(keyv, (_TPAD, _TPAD))
    ridx = jax.lax.broadcasted_iota(jnp.int32, (_TPAD, _TPAD), 0)
    cidx = jax.lax.broadcasted_iota(jnp.int32, (_TPAD, _TPAD), 1)
    loser = jnp.max(jnp.where((kT == kB) & (ridx > cidx), 1.0, 0.0),
                    axis=0, keepdims=True)
    wrow[...] = validv * (1.0 - loser)
    pltpu.make_async_copy(wrow, swin, sem_w).start()

    li = jax.lax.broadcasted_iota(jnp.int32, (_NHS, _NL), 1)
    si = jax.lax.broadcasted_iota(jnp.int32, (_NHS, _NL), 0)
    coli = (li & 63).astype(f32)
    rowi = (si * 2 + (li >> 6)).astype(f32)
    zeros = jnp.zeros((_NHS, _NL), f32)

    lconf = jnp.zeros((1, _NL), f32)
    for a in range(_NUM_ANCHORS):
        af = f32(a)
        sx = jax.nn.sigmoid(raw_blk[0, a, 0])
        sy = jax.nn.sigmoid(raw_blk[0, a, 1])
        pw = jnp.exp(raw_blk[0, a, 2]) * _AW[a]
        ph = jnp.exp(raw_blk[0, a, 3]) * _AH[a]
        px = sx + coli
        py = sy + rowi
        pxl = px - 0.5 * pw
        pxr = px + 0.5 * pw
        pyt = py - 0.5 * ph
        pyb = py + 0.5 * ph
        parea375 = (0.375 * pw) * ph
        sil = zeros > 1.0
        maskc = zeros
        tcfc = zeros
        for t in range(_MAX_GT):
            ox = jnp.minimum(pxr, sc[_R_GXR, t]) - jnp.maximum(pxl, sc[_R_GXL, t])
            oy = jnp.minimum(pyb, sc[_R_GYB, t]) - jnp.maximum(pyt, sc[_R_GYT, t])
            inter = jnp.maximum(ox, 0.0) * oy
            sil = sil | (inter > parea375 + sc[_R_G375, t])
            ii_m = jnp.where(sc[_R_BNV, t] == af, sc[_R_II, t], f32(-1.0))
            hit = (rowi == sc[_R_JJ, t]) & (coli == ii_m)
            maskc = jnp.where(hit, 1.0, maskc)
            tcfc = jnp.where(hit, sc[_R_TCONF, t], tcfc)
        conf = jax.nn.sigmoid(raw_blk[0, a, 4])
        scale = jnp.where(maskc > 0, _OBJECT_SCALE, jnp.where(sil, 0.0, 1.0))
        dconf = scale * (conf - tcfc)
        lconf = lconf + jnp.sum(dconf * dconf, axis=0, keepdims=True)

    pltpu.make_async_copy(wrow, swin, sem_w).wait()
    for t in range(_MAX_GT):
        pltpu.make_async_copy(gbuf.at[t], gbuf.at[t], sem_g).wait()

    lane64 = jax.lax.broadcasted_iota(jnp.int32, (1, _NW), 1).astype(f32)
    si80 = jax.lax.broadcasted_iota(jnp.int32, (_NUM_CLASSES, _NW), 0).astype(f32)
    lx = jnp.zeros((1, _NW), f32)
    ly = jnp.zeros((1, _NW), f32)
    lw = jnp.zeros((1, _NW), f32)
    lh = jnp.zeros((1, _NW), f32)
    lcls = jnp.zeros((1, _NW), f32)
    for t in range(_MAX_GT):
        wv = swin[0, t]
        ii_g = jnp.where(wv > 0, sc[_R_II, t], f32(-1.0))
        lm = lane64 == ii_g
        sxr = jax.nn.sigmoid(gbuf[t, 0:1, :])
        syr = jax.nn.sigmoid(gbuf[t, 1:2, :])
        rwr = gbuf[t, 2:3, :]
        rhr = gbuf[t, 3:4, :]
        lx = lx + jnp.where(lm, (sxr - sc[_R_TX, t]) ** 2, 0.0)
        ly = ly + jnp.where(lm, (syr - sc[_R_TY, t]) ** 2, 0.0)
        lw = lw + jnp.where(lm, (rwr - sc[_R_TW, t]) ** 2, 0.0)
        lh = lh + jnp.where(lm, (rhr - sc[_R_TH, t]) ** 2, 0.0)
        gcls = gbuf[t, 5:5 + _NUM_CLASSES, :]
        lse = jnp.log(jnp.sum(jnp.exp(gcls), axis=0, keepdims=True))
        pick = jnp.sum(jnp.where(si80 == sc[_R_TCLS, t], gcls, 0.0),
                       axis=0, keepdims=True)
        lcls = lcls + jnp.where(lm, (lse - pick) - _LOG80, 0.0)

    def pad128(r):
        return jnp.concatenate([r, jnp.zeros((1, _NL - _NW), f32)], axis=1)

    ccls = pad128(lcls) + f32(_NUM_ANCHORS * _NH * _NW * _LOG80 / _NL)

    stacked = jnp.concatenate(
        [pad128(lx), pad128(ly), pad128(lw), pad128(lh), lconf, ccls,
         validv, jnp.zeros((1, _NL), f32)], axis=0)
    part[0] = stacked


def kernel(output, target):
    nB = output.shape[0]
    out5 = output.reshape(nB, _NUM_ANCHORS, 5 + _NUM_CLASSES, _NH, _NW)
    raw = out5[:, :, :5].reshape(nB, _NUM_ANCHORS, 5, _NHS, _NL)
    tgt = target.reshape(nB, _MAX_GT, 5).transpose(0, 2, 1)
    tgt = jnp.pad(tgt, ((0, 0), (0, 0), (0, _TPAD - _MAX_GT)))
    tgt_flat = tgt.reshape(nB, 5 * _TPAD)

    part = pl.pallas_call(
        _region_loss_kernel,
        out_shape=jax.ShapeDtypeStruct((nB, 8, _NL), jnp.float32),
        grid=(nB,),
        in_specs=[
            pl.BlockSpec(memory_space=pltpu.SMEM),
            pl.BlockSpec((1, 5, _TPAD), lambda b: (b, 0, 0)),
            pl.BlockSpec((1, _NUM_ANCHORS, 5, _NHS, _NL), lambda b: (b, 0, 0, 0, 0)),
            pl.BlockSpec(memory_space=pl.ANY),
        ],
        out_specs=pl.BlockSpec((1, 8, _NL), lambda b: (b, 0, 0)),
        scratch_shapes=[
            pltpu.SMEM((_N_ROWS, _TPAD), jnp.float32),
            pltpu.SMEM((1, _TPAD), jnp.float32),
            pltpu.VMEM((1, _TPAD), jnp.float32),
            pltpu.VMEM((_MAX_GT, 5 + _NUM_CLASSES, _NW), jnp.float32),
            pltpu.SemaphoreType.DMA,
            pltpu.SemaphoreType.DMA,
        ],
        compiler_params=pltpu.CompilerParams(
            dimension_semantics=("parallel",),
        ),
        name="region_loss_sparse",
    )(tgt_flat, tgt, raw, out5)

    sums = jnp.sum(part, axis=(0, 2))
    ngt = sums[6]
    return (sums[0] + sums[1] + sums[2] + sums[3] + sums[4] + sums[5]) / ngt
